# baseline (speedup 1.0000x reference)
import jax
import jax.numpy as jnp
from jax import lax
from jax.experimental import pallas as pl
from jax.experimental.pallas import tpu as pltpu
from jax.experimental.pallas import tpu_sc as plsc

_BSZ = 4
_SEQ = 4096
_DIM = 1024
_NC = 2
_NS = 16
_NW = _NC * _NS
_CHUNK = 32
_NRING = 3
_SEQ_PER_W = _SEQ // _NW
_N_SUB = _SEQ_PER_W // _CHUNK
_L = 16


def _body(inp_hbm, w_hbm, out_hbm, wbufs, tokbuf, cidx, ridx, rowsbuf,
          load_sem, scat_sem, tok_sem, gat_sem, rare_sem):
    wid = lax.axis_index("s") * _NC + lax.axis_index("c")
    base0 = wid * _SEQ_PER_W

    for b in range(_BSZ):
        pltpu.make_async_copy(
            inp_hbm.at[pl.ds(b * _SEQ + base0, _SEQ_PER_W)],
            tokbuf.at[b], tok_sem).start()

    def start_gather(s):
        slot = s % _NRING
        for g in range(_CHUNK // _L):
            iota = lax.broadcasted_iota(jnp.int32, (_L,), 0)
            cidx[slot, pl.ds(g * _L, _L)] = (
                base0 + s * _CHUNK + 1 + g * _L + iota)
        pltpu.make_async_copy(w_hbm.at[cidx.at[slot]], wbufs.at[slot],
                              load_sem).start()

    start_gather(0)
    start_gather(1)

    for b in range(_BSZ):
        pltpu.make_async_copy(
            inp_hbm.at[pl.ds(0, _SEQ_PER_W)], tokbuf.at[0], tok_sem).wait()

    out_slot = out_hbm.at[pl.ds(0, _CHUNK)]
    out_half = out_hbm.at[pl.ds(0, _L)]
    commons = []
    for s in range(_N_SUB):
        wbuf = wbufs.at[s % _NRING]

        pltpu.make_async_copy(w_hbm.at[cidx.at[s % _NRING]], wbuf,
                              load_sem).wait()

        off = s * _CHUNK
        base = base0 + off
        common = []
        for b in range(_BSZ):
            npad = jnp.int32(0)
            for g in range(_CHUNK // _L):
                tok = tokbuf[b, pl.ds(off + g * _L, _L)]
                npad = npad + jnp.sum(jnp.where(tok == 0, 1, 0))
            is_common = npad == 0
            common.append(is_common)
            flat = b * _SEQ + base

            @pl.when(is_common)
            def _():
                pltpu.make_async_copy(wbuf, out_hbm.at[pl.ds(flat, _CHUNK)],
                                      scat_sem).start()

            @pl.when(jnp.logical_not(is_common))
            def _():
                for h in range(_CHUNK // _L):
                    tok = tokbuf[b, pl.ds(off + h * _L, _L)]
                    iota = lax.broadcasted_iota(jnp.int32, (_L,), 0)
                    ridx[...] = jnp.where(
                        tok == 0, 0, base + 1 + h * _L + iota)
                    pltpu.make_async_copy(w_hbm.at[ridx], rowsbuf,
                                          gat_sem).start()
                    pltpu.make_async_copy(w_hbm.at[ridx], rowsbuf,
                                          gat_sem).wait()
                    pltpu.make_async_copy(
                        rowsbuf, out_hbm.at[pl.ds(flat + h * _L, _L)],
                        rare_sem).start()
                    pltpu.make_async_copy(rowsbuf, out_half, rare_sem).wait()

        commons.append(common)

        if s + 2 < _N_SUB:
            if s >= 1:
                for b in range(_BSZ):
                    @pl.when(commons[s - 1][b])
                    def _():
                        pltpu.make_async_copy(wbufs.at[(s - 1) % _NRING],
                                              out_slot, scat_sem).wait()
            start_gather(s + 2)

    for s in range(max(0, _N_SUB - 3), _N_SUB):
        for b in range(_BSZ):
            @pl.when(commons[s][b])
            def _():
                pltpu.make_async_copy(wbufs.at[s % _NRING], out_slot,
                                      scat_sem).wait()


@jax.jit
def _sc_embed(inp_flat, weights):
    mesh = plsc.VectorSubcoreMesh(
        core_axis_name="c", subcore_axis_name="s",
        num_cores=_NC, num_subcores=_NS,
    )
    return pl.kernel(
        _body,
        out_type=jax.ShapeDtypeStruct((_BSZ * _SEQ, _DIM), jnp.float32),
        mesh=mesh,
        scratch_types=[
            pltpu.VMEM((_NRING, _CHUNK, _DIM), jnp.float32),
            pltpu.VMEM((_BSZ, _SEQ_PER_W), jnp.int32),
            pltpu.VMEM((_NRING, _CHUNK), jnp.int32),
            pltpu.VMEM((_L,), jnp.int32),
            pltpu.VMEM((_L, _DIM), jnp.float32),
            pltpu.SemaphoreType.DMA,
            pltpu.SemaphoreType.DMA,
            pltpu.SemaphoreType.DMA,
            pltpu.SemaphoreType.DMA,
            pltpu.SemaphoreType.DMA,
        ],
        compiler_params=pltpu.CompilerParams(
            needs_layout_passes=False, skip_device_barrier=True
        ),
    )(inp_flat, weights)


def kernel(input, weights):
    out = _sc_embed(input.reshape(-1), weights)
    return out.reshape(_BSZ, _SEQ, _DIM)
